# baseline (speedup 1.0000x reference)
import jax
import jax.numpy as jnp
from jax import lax
from jax.experimental import pallas as pl

VOCAB = 1000
MAXLEN = 512
WIDTH = VOCAB + MAXLEN
RB = 512


def _body(x_ref, o_ref):
    i = pl.program_id(0)
    xv = x_ref[...]
    row = lax.broadcasted_iota(jnp.int32, (RB, 1), 0) + i * RB
    posv = lax.rem(row, 50)
    colv = lax.broadcasted_iota(jnp.int32, (RB, VOCAB), 1)
    colp = lax.broadcasted_iota(jnp.int32, (RB, MAXLEN), 1)
    o_ref[:, :VOCAB] = jnp.where(colv == xv, 1.0, 0.0).astype(jnp.float32)
    o_ref[:, VOCAB:] = jnp.where(colp == posv, 1.0, 0.0).astype(jnp.float32)


def kernel(x):
    b, s = x.shape
    n = b * s
    x2 = x.reshape(n, 1)
    out = pl.pallas_call(
        _body,
        grid=(n // RB,),
        in_specs=[pl.BlockSpec((RB, 1), lambda i: (i, 0))],
        out_specs=pl.BlockSpec((RB, WIDTH), lambda i: (i, 0)),
        out_shape=jax.ShapeDtypeStruct((n, WIDTH), jnp.float32),
    )(x2)
    return out.reshape(b, s, WIDTH)


# speedup vs baseline: 1.0425x; 1.0425x over previous
import jax
import jax.numpy as jnp
from jax import lax
from jax.experimental import pallas as pl

VOCAB = 1000
MAXLEN = 512
WIDTH = VOCAB + MAXLEN
RB = 3200


def _body(x_ref, o_ref):
    i = pl.program_id(0)
    xv = x_ref[...]
    row = lax.broadcasted_iota(jnp.int32, (RB, 1), 0) + i * RB
    posv = lax.rem(row, 50)
    colv = lax.broadcasted_iota(jnp.int32, (RB, VOCAB), 1)
    colp = lax.broadcasted_iota(jnp.int32, (RB, MAXLEN), 1)
    o_ref[:, :VOCAB] = jnp.where(colv == xv, 1.0, 0.0).astype(jnp.float32)
    o_ref[:, VOCAB:] = jnp.where(colp == posv, 1.0, 0.0).astype(jnp.float32)


def kernel(x):
    b, s = x.shape
    n = b * s
    x2 = x.reshape(n, 1)
    out = pl.pallas_call(
        _body,
        grid=(n // RB,),
        in_specs=[pl.BlockSpec((RB, 1), lambda i: (i, 0))],
        out_specs=pl.BlockSpec((RB, WIDTH), lambda i: (i, 0)),
        out_shape=jax.ShapeDtypeStruct((n, WIDTH), jnp.float32),
    )(x2)
    return out.reshape(b, s, WIDTH)


# speedup vs baseline: 1.6553x; 1.5878x over previous
import jax
import jax.numpy as jnp
from jax import lax
from jax.experimental import pallas as pl
from jax.experimental.pallas import tpu as pltpu

VOCAB = 1000
MAXLEN = 512
WIDTH = VOCAB + MAXLEN
BB = 16
NBUF = 4


def _body(x_ref, o_ref, buf, sems):
    i = pl.program_id(0)
    nstep = pl.num_programs(0)
    slot = lax.rem(i, NBUF)

    @pl.when(i >= NBUF)
    def _wait_prev():
        prev = i - NBUF
        pltpu.make_async_copy(
            buf.at[slot], o_ref.at[pl.ds(prev * BB, BB), :, :], sems.at[slot]
        ).wait()

    xv = x_ref[...]
    shape = (BB, x_ref.shape[1], WIDTH)
    col = lax.broadcasted_iota(jnp.int32, shape, 2)
    pos = lax.broadcasted_iota(jnp.int32, shape, 1) + VOCAB
    hit = (col == xv[:, :, None]) | (col == pos)
    buf[slot] = jnp.where(hit, 1.0, 0.0).astype(jnp.float32)

    pltpu.make_async_copy(
        buf.at[slot], o_ref.at[pl.ds(i * BB, BB), :, :], sems.at[slot]
    ).start()

    @pl.when(i == nstep - 1)
    def _drain():
        for j in range(NBUF - 1):
            s = lax.rem(i + 1 + j, NBUF)
            step = i + 1 + j - NBUF
            pltpu.make_async_copy(
                buf.at[s], o_ref.at[pl.ds(step * BB, BB), :, :], sems.at[s]
            ).wait()
        pltpu.make_async_copy(
            buf.at[slot], o_ref.at[pl.ds(i * BB, BB), :, :], sems.at[slot]
        ).wait()


def kernel(x):
    b, s = x.shape
    return pl.pallas_call(
        _body,
        grid=(b // BB,),
        in_specs=[pl.BlockSpec((BB, s), lambda i: (i, 0))],
        out_specs=pl.BlockSpec(memory_space=pl.ANY),
        out_shape=jax.ShapeDtypeStruct((b, s, WIDTH), jnp.float32),
        scratch_shapes=[
            pltpu.VMEM((NBUF, BB, s, WIDTH), jnp.float32),
            pltpu.SemaphoreType.DMA((NBUF,)),
        ],
    )(x)


# speedup vs baseline: 6.9206x; 4.1809x over previous
import jax
import jax.numpy as jnp
from jax import lax
from jax.experimental import pallas as pl

VOCAB = 1000
MAXLEN = 512
WIDTH = VOCAB + MAXLEN
SB = 2


def _body(xt_ref, o_ref):
    i = pl.program_id(0)
    xt = xt_ref[:, 0, :]
    shape_v = (SB, VOCAB, xt_ref.shape[2])
    shape_p = (SB, MAXLEN, xt_ref.shape[2])
    colv = lax.broadcasted_iota(jnp.int32, shape_v, 1)
    o_ref[:, :VOCAB, :] = jnp.where(colv == xt[:, None, :], 1.0, 0.0).astype(
        jnp.float32
    )
    colp = lax.broadcasted_iota(jnp.int32, shape_p, 1)
    s_idx = lax.broadcasted_iota(jnp.int32, shape_p, 0) + i * SB
    o_ref[:, VOCAB:, :] = jnp.where(colp == s_idx, 1.0, 0.0).astype(jnp.float32)


def kernel(x):
    b, s = x.shape
    xt = x.T.reshape(s, 1, b)
    out = pl.pallas_call(
        _body,
        grid=(s // SB,),
        in_specs=[pl.BlockSpec((SB, 1, b), lambda i: (i, 0, 0))],
        out_specs=pl.BlockSpec((SB, WIDTH, b), lambda i: (i, 0, 0)),
        out_shape=jax.ShapeDtypeStruct((s, WIDTH, b), jnp.float32),
    )(xt)
    return out.transpose(2, 0, 1)
